# baseline (speedup 1.0000x reference)
import functools

import jax
import jax.numpy as jnp
from jax import lax
from jax.experimental import pallas as pl
from jax.experimental.pallas import tpu as pltpu
from jax.experimental.pallas import tpu_sc as plsc

PATCH = 16
L = 16
NC, NS = 2, 16
NW = NC * NS
CH = 3072
GS = 256
NSTR = CH // GS


@functools.lru_cache(maxsize=None)
def _build(B, C, H, W):
    nh = W // PATCH
    nv = H // PATCH
    P = nv * nh
    R = B * C * H * nh
    rbc = H * nh
    rows_per_tile = R // NW
    assert rows_per_tile % CH == 0 and rbc % CH == 0
    nchunk = rows_per_tile // CH

    mesh = plsc.VectorSubcoreMesh(core_axis_name="c", subcore_axis_name="s")

    @functools.partial(
        pl.kernel,
        mesh=mesh,
        compiler_params=pltpu.CompilerParams(
            needs_layout_passes=False, use_tc_tiling_on_sc=False),
        out_type=jax.ShapeDtypeStruct((R, L), jnp.float32),
        scratch_types=[
            pltpu.VMEM((B * P,), jnp.int32),
            pltpu.VMEM((rbc,), jnp.int32),
            pltpu.VMEM((CH,), jnp.int32),
            pltpu.VMEM((CH,), jnp.int32),
            pltpu.VMEM((CH, L), jnp.float32),
            pltpu.VMEM((CH, L), jnp.float32),
            pltpu.SemaphoreType.DMA,
            pltpu.SemaphoreType.DMA,
            pltpu.SemaphoreType.DMA,
        ],
    )
    def shuffle(img_hbm, perms_hbm, out_hbm, perms_v, rel_v,
                idx0, idx1, rows0, rows1, gsem, wsem0, wsem1):
        wid = lax.axis_index("s") * NC + lax.axis_index("c")
        base = wid * rows_per_tile
        b_img = base // (C * rbc)
        pltpu.sync_copy(perms_hbm, perms_v)
        iota = lax.iota(jnp.int32, L)
        idx = (idx0, idx1)
        rows = (rows0, rows1)
        wsem = (wsem0, wsem1)

        def rel_body(m, carry):
            j = m * L + iota
            ph = lax.rem(j, nh)
            h = lax.div(j, nh)
            i = lax.rem(h, PATCH)
            pv = lax.div(h, PATCH)
            pidx = b_img * P + pv * nh + ph
            s = plsc.load_gather(perms_v, [pidx])
            sv = lax.div(s, nh)
            sh = lax.rem(s, nh)
            rel_v[pl.ds(m * L, L)] = (sv * PATCH + i) * nh + sh
            return carry

        lax.fori_loop(0, rbc // L, rel_body, 0)

        def gen_idx(cc, idx_ref):
            r0 = base + cc * CH
            plane = lax.div(r0, rbc) * rbc
            off = lax.rem(r0, rbc)

            def idx_body(m, carry):
                idx_ref[pl.ds(m * L, L)] = (
                    rel_v[pl.ds(off + m * L, L)] + plane)
                return carry

            lax.fori_loop(0, CH // L, idx_body, 0)

        gen_idx(0, idx0)

        def pair_body(g2, carry):
            for sl in range(2):
                gg = g2 * 2 + sl
                r0 = base + gg * CH

                @pl.when(gg >= 2)
                def _():
                    pltpu.make_async_copy(
                        rows[sl], out_hbm.at[pl.ds(r0, CH)], wsem[sl]).wait()

                for j in range(NSTR):
                    pltpu.make_async_copy(
                        img_hbm.at[idx[sl].at[pl.ds(j * GS, GS)]],
                        rows[sl].at[pl.ds(j * GS, GS)],
                        gsem,
                    ).start()

                gen_idx(jnp.minimum(gg + 1, nchunk - 1), idx[1 - sl])

                for j in range(NSTR):
                    pltpu.make_async_copy(
                        img_hbm.at[idx[sl].at[pl.ds(j * GS, GS)]],
                        rows[sl].at[pl.ds(j * GS, GS)],
                        gsem,
                    ).wait()

                pltpu.make_async_copy(
                    rows[sl], out_hbm.at[pl.ds(r0, CH)], wsem[sl]).start()
            return carry

        lax.fori_loop(0, nchunk // 2, pair_body, 0)

        for sl in range(2):
            pltpu.make_async_copy(
                rows[sl], out_hbm.at[pl.ds(base, CH)], wsem[sl]).wait()

    return shuffle


def kernel(img, perms):
    B, C, H, W = img.shape
    nh = W // PATCH
    R = B * C * H * nh
    table = img.reshape(R, L)
    out = _build(B, C, H, W)(table, perms.reshape(-1).astype(jnp.int32))
    return out.reshape(B, C, H, W)


# speedup vs baseline: 1.0031x; 1.0031x over previous
import functools

import jax
import jax.numpy as jnp
from jax import lax
from jax.experimental import pallas as pl
from jax.experimental.pallas import tpu as pltpu
from jax.experimental.pallas import tpu_sc as plsc

PATCH = 16
L = 16
NC, NS = 2, 16
NW = NC * NS
CH = 3072
GS = 256
NSTR = CH // GS


@functools.lru_cache(maxsize=None)
def _build(B, C, H, W):
    nh = W // PATCH
    nv = H // PATCH
    P = nv * nh
    R = B * C * H * nh
    rbc = H * nh
    rows_per_tile = R // NW
    assert rows_per_tile % CH == 0 and rbc % CH == 0
    nchunk = rows_per_tile // CH

    mesh = plsc.VectorSubcoreMesh(core_axis_name="c", subcore_axis_name="s")

    @functools.partial(
        pl.kernel,
        mesh=mesh,
        compiler_params=pltpu.CompilerParams(
            needs_layout_passes=False, use_tc_tiling_on_sc=False),
        out_type=jax.ShapeDtypeStruct((R, L), jnp.float32),
        scratch_types=[
            pltpu.VMEM((B * P,), jnp.int32),
            pltpu.VMEM((rbc,), jnp.int32),
            pltpu.VMEM((CH, L), jnp.float32),
            pltpu.VMEM((CH, L), jnp.float32),
            pltpu.SemaphoreType.DMA,
            pltpu.SemaphoreType.DMA,
            pltpu.SemaphoreType.DMA,
        ],
    )
    def shuffle(img_hbm, perms_hbm, out_hbm, perms_v, rel_v,
                rows0, rows1, gsem, wsem0, wsem1):
        wid = lax.axis_index("s") * NC + lax.axis_index("c")
        base = wid * rows_per_tile
        b_img = base // (C * rbc)
        pltpu.sync_copy(perms_hbm, perms_v)
        iota = lax.iota(jnp.int32, L)
        rows = (rows0, rows1)
        wsem = (wsem0, wsem1)

        def rel_body(m, carry):
            j = m * L + iota
            ph = lax.rem(j, nh)
            h = lax.div(j, nh)
            i = lax.rem(h, PATCH)
            pv = lax.div(h, PATCH)
            pidx = b_img * P + pv * nh + ph
            s = plsc.load_gather(perms_v, [pidx])
            sv = lax.div(s, nh)
            sh = lax.rem(s, nh)
            rel_v[pl.ds(m * L, L)] = (sv * PATCH + i) * nh + sh
            return carry

        lax.fori_loop(0, rbc // L, rel_body, 0)

        def pair_body(g2, carry):
            for sl in range(2):
                gg = g2 * 2 + sl
                r0 = base + gg * CH
                plane = lax.div(r0, rbc) * rbc
                off = lax.rem(r0, rbc)
                table = img_hbm.at[pl.ds(plane, rbc)]

                @pl.when(gg >= 2)
                def _():
                    pltpu.make_async_copy(
                        rows[sl], out_hbm.at[pl.ds(r0, CH)], wsem[sl]).wait()

                for j in range(NSTR):
                    pltpu.make_async_copy(
                        table.at[rel_v.at[pl.ds(off + j * GS, GS)]],
                        rows[sl].at[pl.ds(j * GS, GS)],
                        gsem,
                    ).start()

                for j in range(NSTR):
                    pltpu.make_async_copy(
                        table.at[rel_v.at[pl.ds(off + j * GS, GS)]],
                        rows[sl].at[pl.ds(j * GS, GS)],
                        gsem,
                    ).wait()

                pltpu.make_async_copy(
                    rows[sl], out_hbm.at[pl.ds(r0, CH)], wsem[sl]).start()
            return carry

        lax.fori_loop(0, nchunk // 2, pair_body, 0)

        for sl in range(2):
            pltpu.make_async_copy(
                rows[sl], out_hbm.at[pl.ds(base, CH)], wsem[sl]).wait()

    return shuffle


def kernel(img, perms):
    B, C, H, W = img.shape
    nh = W // PATCH
    R = B * C * H * nh
    table = img.reshape(R, L)
    out = _build(B, C, H, W)(table, perms.reshape(-1).astype(jnp.int32))
    return out.reshape(B, C, H, W)
